# baseline (speedup 1.0000x reference)
import jax
import jax.numpy as jnp
from jax.experimental import pallas as pl
from jax.experimental.pallas import tpu as pltpu


def _copy_body(x_ref, o_ref, sem):
    copy = pltpu.make_async_copy(x_ref, o_ref, sem)
    copy.start()
    copy.wait()


def kernel(x):
    return pl.pallas_call(
        _copy_body,
        in_specs=[pl.BlockSpec(memory_space=pl.ANY)],
        out_specs=pl.BlockSpec(memory_space=pl.ANY),
        out_shape=jax.ShapeDtypeStruct(x.shape, x.dtype),
        scratch_shapes=[pltpu.SemaphoreType.DMA],
    )(x)


# speedup vs baseline: 14.5479x; 14.5479x over previous
import jax
import jax.numpy as jnp
from jax.experimental import pallas as pl
from jax.experimental.pallas import tpu as pltpu


def _copy_block(x_ref, o_ref):
    o_ref[...] = x_ref[...]


def kernel(x):
    B, N, c, num, dim = x.shape
    NJ = 5
    block = (1, N // NJ, c, num, dim)
    return pl.pallas_call(
        _copy_block,
        grid=(B, NJ),
        in_specs=[pl.BlockSpec(block, lambda i, j: (i, j, 0, 0, 0))],
        out_specs=pl.BlockSpec(block, lambda i, j: (i, j, 0, 0, 0)),
        out_shape=jax.ShapeDtypeStruct(x.shape, x.dtype),
    )(x)
